# baseline (speedup 1.0000x reference)
import functools

import jax
import jax.numpy as jnp
from jax import lax
from jax.experimental import pallas as pl
from jax.experimental.pallas import tpu as pltpu
from jax.experimental.pallas import tpu_sc as plsc

V = 1_000_000
H = 64
T = 200
SQ = 16384
NC, NS = 2, 16
NW = NC * NS
UNITS = T * (SQ // 128)
UPW = UNITS // NW
VCH = V // 128
VREM_AT = VCH * 128
S1_IT = VCH // NS + 1


def _make_gather():
    mesh = plsc.VectorSubcoreMesh(core_axis_name="c", subcore_axis_name="s")

    @functools.partial(
        pl.kernel,
        mesh=mesh,
        out_type=[
            jax.ShapeDtypeStruct((T, H, SQ), jnp.float32),
            jax.ShapeDtypeStruct((NC, V, 128), jnp.float32),
        ],
        scratch_types=[
            pltpu.VMEM((3, H, 128), jnp.float32),
            pltpu.VMEM((2, 128, 128), jnp.float32),
            pltpu.VMEM((2, 8, 128), jnp.int32),
            pltpu.VMEM((3, 128, 128), jnp.float32),
            pltpu.VMEM((2, H, 128), jnp.float32),
            pltpu.SemaphoreType.DMA,
            pltpu.SemaphoreType.DMA,
            pltpu.SemaphoreType.DMA,
            pltpu.SemaphoreType.DMA,
        ],
        compiler_params=pltpu.CompilerParams(
            use_tc_tiling_on_sc=True, needs_layout_passes=False
        ),
    )
    def gather_kernel(wT_hbm, xT_hbm, wtail_hbm, out_hbm, tbl_hbm, raw1, tr1,
                      idxb, raw2, tr2, semr, semw, semg, semo):
        core = lax.axis_index("c")
        sid = lax.axis_index("s")
        wid = sid * NC + core
        iota = lax.iota(jnp.int32, 16)

        def s1_read(j):
            ch = sid + j * NS
            pltpu.async_copy(
                wT_hbm.at[:, pl.ds(pl.multiple_of(ch * 128, 128), 128)],
                raw1.at[lax.rem(j, 3)], semr,
            )

        def s1_transpose(bsrc, bdst, nrows, coff):
            def body(v, _):
                for g in range(4):
                    vec = plsc.load_gather(
                        raw1.at[bsrc],
                        [g * 16 + iota, jnp.full((16,), coff + v, jnp.int32)],
                    )
                    tr1[bdst, v, pl.ds(g * 16, 16)] = vec
                return _
            lax.fori_loop(0, nrows, body, None)

        s1_read(0)
        s1_read(1)

        def s1_step(j, _):
            ch = sid + j * NS

            @pl.when((j >= 2) & (sid + (j - 2) * NS < VCH))
            def _():
                pltpu.make_async_copy(
                    tr1.at[0], tbl_hbm.at[core, pl.ds(0, 128)], semw,
                ).wait()

            @pl.when(sid + (j + 2) * NS < VCH)
            def _():
                s1_read(j + 2)

            @pl.when(ch < VCH)
            def _():
                pltpu.make_async_copy(
                    wT_hbm.at[:, pl.ds(0, 128)], raw1.at[0], semr,
                ).wait()
                s1_transpose(lax.rem(j, 3), lax.rem(j, 2), 128, 0)
                pltpu.async_copy(
                    tr1.at[lax.rem(j, 2)],
                    tbl_hbm.at[core, pl.ds(pl.multiple_of(ch * 128, 128), 128)],
                    semw,
                )

            @pl.when(ch == VCH)
            def _():
                pltpu.sync_copy(
                    wtail_hbm,
                    tr1.at[lax.rem(j, 2), pl.ds(0, 64), :],
                )
                pltpu.sync_copy(
                    tr1.at[lax.rem(j, 2), pl.ds(0, 64), :],
                    tbl_hbm.at[core, pl.ds(VREM_AT, 64)],
                )
            return _

        lax.fori_loop(0, S1_IT, s1_step, None)

        def s1_drain_w():
            pltpu.make_async_copy(
                tr1.at[0], tbl_hbm.at[core, pl.ds(0, 128)], semw,
            ).wait()

        s1_drain_w()

        @pl.when(sid + (S1_IT - 1) * NS < VCH)
        def _():
            s1_drain_w()

        plsc.subcore_barrier()

        def unit_addr(u):
            blk = wid * (UPW // 8) + lax.div(u, 8)
            return lax.div(blk, 128), lax.rem(u, 8), lax.rem(blk, 128)

        def fire_unit(u):
            tb, tloc, sb = unit_addr(u)

            @pl.when(lax.rem(u, 8) == 0)
            def _():
                pltpu.sync_copy(
                    xT_hbm.at[tb, :, pl.ds(pl.multiple_of(sb * 128, 128), 128)],
                    idxb.at[lax.rem(lax.div(u, 8), 2)],
                )

            pltpu.async_copy(
                tbl_hbm.at[core].at[
                    idxb.at[lax.rem(lax.div(u, 8), 2), tloc]
                ],
                raw2.at[lax.rem(u, 3)], semg,
            )

        def s3_transpose(bsrc, bdst):
            def body(h, _):
                for g in range(8):
                    vec = plsc.load_gather(
                        raw2.at[bsrc],
                        [g * 16 + iota, jnp.full((16,), h, jnp.int32)],
                    )
                    tr2[bdst, h, pl.ds(g * 16, 16)] = vec
                return _
            lax.fori_loop(0, H, body, None)

        fire_unit(0)
        fire_unit(1)

        def s23_step(u, _):
            @pl.when(u >= 4)
            def _():
                pltpu.make_async_copy(
                    tr2.at[0], out_hbm.at[0, :, pl.ds(0, 128)], semo,
                ).wait()

            pltpu.make_async_copy(
                tbl_hbm.at[core, pl.ds(0, 128)], raw2.at[0], semg,
            ).wait()

            @pl.when(u < UPW)
            def _():
                fire_unit(u)

            s3_transpose(lax.rem(u - 2, 3), lax.rem(u, 2))
            tb, tloc, sb = unit_addr(u - 2)
            pltpu.async_copy(
                tr2.at[lax.rem(u, 2)],
                out_hbm.at[tb * 8 + tloc, :, pl.ds(pl.multiple_of(sb * 128, 128), 128)], semo,
            )
            return _

        lax.fori_loop(2, UPW + 2, s23_step, None)

        for _ in range(2):
            pltpu.make_async_copy(
                tr2.at[0], out_hbm.at[0, :, pl.ds(0, 128)], semo,
            ).wait()

    return gather_kernel


_gather = _make_gather()


def kernel(x, weight):
    xT3 = x.astype(jnp.int32).T.reshape(T // 8, 8, SQ)
    wtail = jnp.pad(weight[VREM_AT:], ((0, 0), (0, 128 - H)))
    outT, _ = _gather(weight.T, xT3, wtail)
    return jnp.transpose(outT, (2, 0, 1))


# speedup vs baseline: 1.0027x; 1.0027x over previous
import functools

import jax
import jax.numpy as jnp
from jax import lax
from jax.experimental import pallas as pl
from jax.experimental.pallas import tpu as pltpu
from jax.experimental.pallas import tpu_sc as plsc

V = 1_000_000
H = 64
T = 200
SQ = 16384
NC, NS = 2, 16
NW = NC * NS
UNITS = T * (SQ // 128)
UPW = UNITS // NW
VCH = V // 128
VREM_AT = VCH * 128
S1_IT = VCH // NS + 1


def _make_gather():
    mesh = plsc.VectorSubcoreMesh(core_axis_name="c", subcore_axis_name="s")

    @functools.partial(
        pl.kernel,
        mesh=mesh,
        out_type=[
            jax.ShapeDtypeStruct((T, H, SQ), jnp.float32),
            jax.ShapeDtypeStruct((NC, V, 128), jnp.float32),
        ],
        scratch_types=[
            pltpu.VMEM((3, H, 128), jnp.float32),
            pltpu.VMEM((2, 128, 128), jnp.float32),
            pltpu.VMEM((2, 8, 128), jnp.int32),
            pltpu.VMEM((3, 128, 128), jnp.float32),
            pltpu.VMEM((2, H, 128), jnp.float32),
            pltpu.SemaphoreType.DMA,
            pltpu.SemaphoreType.DMA,
            pltpu.SemaphoreType.DMA,
            pltpu.SemaphoreType.DMA,
        ],
        compiler_params=pltpu.CompilerParams(
            use_tc_tiling_on_sc=True, needs_layout_passes=False
        ),
    )
    def gather_kernel(wT_hbm, xT_hbm, wtail_hbm, out_hbm, tbl_hbm, raw1, tr1,
                      idxb, raw2, tr2, semr, semw, semg, semo):
        core = lax.axis_index("c")
        sid = lax.axis_index("s")
        wid = sid * NC + core
        iota = lax.iota(jnp.int32, 16)

        def s1_read(j):
            ch = sid + j * NS
            pltpu.async_copy(
                wT_hbm.at[:, pl.ds(pl.multiple_of(ch * 128, 128), 128)],
                raw1.at[lax.rem(j, 3)], semr,
            )

        rows4 = [(g * 16 + iota) for g in range(4)]
        rows8 = [(g * 16 + iota) for g in range(8)]

        def s1_transpose(bsrc, bdst):
            def body(vo, _):
                for k in range(8):
                    v = vo * 8 + k
                    hs = jnp.full((16,), 0, jnp.int32) + v
                    for g in range(4):
                        vec = plsc.load_gather(raw1.at[bsrc], [rows4[g], hs])
                        tr1[bdst, v, pl.ds(g * 16, 16)] = vec
                return _
            lax.fori_loop(0, 16, body, None)

        s1_read(0)
        s1_read(1)

        def s1_step(j, _):
            ch = sid + j * NS

            @pl.when((j >= 2) & (sid + (j - 2) * NS < VCH))
            def _():
                pltpu.make_async_copy(
                    tr1.at[0], tbl_hbm.at[core, pl.ds(0, 128)], semw,
                ).wait()

            @pl.when(sid + (j + 2) * NS < VCH)
            def _():
                s1_read(j + 2)

            @pl.when(ch < VCH)
            def _():
                pltpu.make_async_copy(
                    wT_hbm.at[:, pl.ds(0, 128)], raw1.at[0], semr,
                ).wait()
                s1_transpose(lax.rem(j, 3), lax.rem(j, 2))
                pltpu.async_copy(
                    tr1.at[lax.rem(j, 2)],
                    tbl_hbm.at[core, pl.ds(pl.multiple_of(ch * 128, 128), 128)],
                    semw,
                )

            @pl.when(ch == VCH)
            def _():
                pltpu.sync_copy(
                    wtail_hbm,
                    tr1.at[lax.rem(j, 2), pl.ds(0, 64), :],
                )
                pltpu.sync_copy(
                    tr1.at[lax.rem(j, 2), pl.ds(0, 64), :],
                    tbl_hbm.at[core, pl.ds(VREM_AT, 64)],
                )
            return _

        lax.fori_loop(0, S1_IT, s1_step, None)

        def s1_drain_w():
            pltpu.make_async_copy(
                tr1.at[0], tbl_hbm.at[core, pl.ds(0, 128)], semw,
            ).wait()

        s1_drain_w()

        @pl.when(sid + (S1_IT - 1) * NS < VCH)
        def _():
            s1_drain_w()

        plsc.subcore_barrier()

        def unit_addr(u):
            blk = wid * (UPW // 8) + lax.div(u, 8)
            return lax.div(blk, 128), lax.rem(u, 8), lax.rem(blk, 128)

        def fire_unit(u):
            tb, tloc, sb = unit_addr(u)

            @pl.when(lax.rem(u, 8) == 0)
            def _():
                pltpu.sync_copy(
                    xT_hbm.at[tb, :, pl.ds(pl.multiple_of(sb * 128, 128), 128)],
                    idxb.at[lax.rem(lax.div(u, 8), 2)],
                )

            pltpu.async_copy(
                tbl_hbm.at[core].at[
                    idxb.at[lax.rem(lax.div(u, 8), 2), tloc]
                ],
                raw2.at[lax.rem(u, 3)], semg,
            )

        def s3_transpose(bsrc, bdst):
            def body(ho, _):
                for k in range(8):
                    h = ho * 8 + k
                    hs = jnp.full((16,), 0, jnp.int32) + h
                    for g in range(8):
                        vec = plsc.load_gather(raw2.at[bsrc], [rows8[g], hs])
                        tr2[bdst, h, pl.ds(g * 16, 16)] = vec
                return _
            lax.fori_loop(0, 8, body, None)

        fire_unit(0)
        fire_unit(1)

        def s23_step(u, _):
            @pl.when(u >= 4)
            def _():
                pltpu.make_async_copy(
                    tr2.at[0], out_hbm.at[0, :, pl.ds(0, 128)], semo,
                ).wait()

            pltpu.make_async_copy(
                tbl_hbm.at[core, pl.ds(0, 128)], raw2.at[0], semg,
            ).wait()

            @pl.when(u < UPW)
            def _():
                fire_unit(u)

            s3_transpose(lax.rem(u - 2, 3), lax.rem(u, 2))
            tb, tloc, sb = unit_addr(u - 2)
            pltpu.async_copy(
                tr2.at[lax.rem(u, 2)],
                out_hbm.at[tb * 8 + tloc, :, pl.ds(pl.multiple_of(sb * 128, 128), 128)], semo,
            )
            return _

        lax.fori_loop(2, UPW + 2, s23_step, None)

        for _ in range(2):
            pltpu.make_async_copy(
                tr2.at[0], out_hbm.at[0, :, pl.ds(0, 128)], semo,
            ).wait()

    return gather_kernel


_gather = _make_gather()


def kernel(x, weight):
    xT3 = x.astype(jnp.int32).T.reshape(T // 8, 8, SQ)
    wtail = jnp.pad(weight[VREM_AT:], ((0, 0), (0, 128 - H)))
    outT, _ = _gather(weight.T, xT3, wtail)
    return jnp.transpose(outT, (2, 0, 1))


# speedup vs baseline: 3.3298x; 3.3209x over previous
import functools

import jax
import jax.numpy as jnp
from jax import lax
from jax.experimental import pallas as pl
from jax.experimental.pallas import tpu as pltpu
from jax.experimental.pallas import tpu_sc as plsc

V = 1_000_000
H = 64
T = 200
SQ = 16384
NC, NS = 2, 16
NW = NC * NS
SPW = SQ // NW
SPLIT = (0, 128), (128, T - 128)


def _make_gather():
    mesh = plsc.VectorSubcoreMesh(core_axis_name="c", subcore_axis_name="s")

    @functools.partial(
        pl.kernel,
        mesh=mesh,
        out_type=jax.ShapeDtypeStruct((SQ, T, 128), jnp.float32),
        scratch_types=[
            pltpu.VMEM((2, 8, T), jnp.int32),
            pltpu.VMEM((3, T, 128), jnp.float32),
            pltpu.SemaphoreType.DMA,
            pltpu.SemaphoreType.DMA,
        ],
        compiler_params=pltpu.CompilerParams(
            use_tc_tiling_on_sc=True, needs_layout_passes=False
        ),
    )
    def gather_kernel(tbl_hbm, x_hbm, out_hbm, idxb, rows, semg, semo):
        core = lax.axis_index("c")
        sid = lax.axis_index("s")
        wid = sid * NC + core
        s0 = wid * SPW

        def fire(u):
            @pl.when(lax.rem(u, 8) == 0)
            def _():
                pltpu.sync_copy(
                    x_hbm.at[pl.ds(pl.multiple_of(s0 + u, 8), 8)],
                    idxb.at[lax.rem(lax.div(u, 8), 2)],
                )
            for off, ln in SPLIT:
                pltpu.async_copy(
                    tbl_hbm.at[
                        idxb.at[lax.rem(lax.div(u, 8), 2), lax.rem(u, 8),
                                pl.ds(off, ln)]
                    ],
                    rows.at[lax.rem(u, 3), pl.ds(off, ln)], semg,
                )

        def wait_gather(b):
            pltpu.make_async_copy(
                tbl_hbm.at[pl.ds(0, T)], rows.at[b], semg
            ).wait()

        def wait_out():
            pltpu.make_async_copy(
                rows.at[0], out_hbm.at[0], semo
            ).wait()

        fire(0)
        fire(1)

        def step(u, _):
            @pl.when(u >= 4)
            def _():
                wait_out()

            wait_gather(lax.rem(u - 2, 3))

            @pl.when(u < SPW)
            def _():
                fire(u)

            pltpu.async_copy(
                rows.at[lax.rem(u - 2, 3)], out_hbm.at[s0 + u - 2], semo,
            )
            return _

        lax.fori_loop(2, SPW + 2, step, None)

        for _ in range(2):
            wait_out()

    return gather_kernel


_gather = _make_gather()


def kernel(x, weight):
    tbl = jnp.pad(weight, ((0, 0), (0, 128 - H)))
    out128 = _gather(tbl, x.astype(jnp.int32))
    return out128[:, :, :H]
